# baseline (speedup 1.0000x reference)
import functools

import jax
import jax.numpy as jnp
from jax import lax
from jax.experimental import pallas as pl
from jax.experimental.pallas import tpu as pltpu
from jax.experimental.pallas import tpu_sc as plsc

VOCAB = 100000
D = 768
SEQ = 50
BATCH = 1024
N_TOK = BATCH * SEQ
LANES = 16
KD = D // LANES

_info = plsc.get_sparse_core_info()
NC, NS = _info.num_cores, _info.num_subcores
NW = NC * NS
TPW = N_TOK // NW
CH = 32
NCHUNK = TPW // CH
POS_STAGE = 56

_F32 = jnp.float32


def _rsqrt16(x):
    xi = lax.bitcast_convert_type(x, jnp.int32)
    yi = jnp.int32(0x5F3759DF) - (xi >> 1)
    y = lax.bitcast_convert_type(yi, _F32)
    for _ in range(4):
        y = y * (1.5 - 0.5 * x * y * y)
    return y


def _body(ids_hbm, tti_hbm, tok_hbm, pos_hbm, seg_hbm, gam_hbm, bet_hbm,
          out_hbm, idx_v, tti_v, pos_v, seg_v, gam_v, bet_v, buf, sem):
    wid = lax.axis_index("s") * NC + lax.axis_index("c")
    wbase = wid * TPW

    pltpu.sync_copy(ids_hbm.at[pl.ds(wbase, TPW)], idx_v)
    pltpu.sync_copy(tti_hbm.at[pl.ds(wbase, TPW)], tti_v)
    pltpu.sync_copy(pos_hbm.at[pl.ds(0, POS_STAGE)], pos_v)
    pltpu.sync_copy(seg_hbm, seg_v)
    pltpu.sync_copy(gam_hbm, gam_v)
    pltpu.sync_copy(bet_hbm, bet_v)

    def fold_pos(i, _):
        p, k = i // KD, i % KD
        pos_v[p, pl.ds(k * LANES, LANES)] = (
            pos_v[p, pl.ds(k * LANES, LANES)]
            + seg_v[0, pl.ds(k * LANES, LANES)])
        return 0

    lax.fori_loop(0, SEQ * KD, fold_pos, 0)

    def fold_seg(k, _):
        seg_v[1, pl.ds(k * LANES, LANES)] = (
            seg_v[1, pl.ds(k * LANES, LANES)]
            - seg_v[0, pl.ds(k * LANES, LANES)])
        return 0

    lax.fori_loop(0, KD, fold_seg, 0)

    zero = jnp.zeros((LANES,), _F32)
    lane = lax.iota(jnp.int32, LANES)

    def chunk_body(j, _):
        pltpu.async_copy(tok_hbm.at[idx_v.at[pl.ds(j * CH, CH)]], buf,
                         sem).wait()

        def row_body(r, _):
            t = j * CH + r
            p = lax.rem(t, SEQ)
            ttv = tti_v[pl.ds((t // LANES) * LANES, LANES)]
            lane_idx = jnp.full((LANES,), lax.rem(r, LANES), jnp.int32)
            ttf = ttv.at[lane_idx].get(mode="promise_in_bounds").astype(_F32)

            def k_pass1(k, carry):
                s, q = carry
                v = (buf[r, pl.ds(k * LANES, LANES)]
                     + pos_v[p, pl.ds(k * LANES, LANES)]
                     + ttf * seg_v[1, pl.ds(k * LANES, LANES)])
                buf[r, pl.ds(k * LANES, LANES)] = v
                return (s + v, q + v * v)

            s, q = lax.fori_loop(0, KD, k_pass1, (zero, zero))
            mean = jnp.sum(s) * (1.0 / D)
            ex2 = jnp.sum(q) * (1.0 / D)
            var = ex2 - mean * mean
            rstd = _rsqrt16(jnp.full((LANES,), var + 1e-5, _F32))
            mvec = jnp.full((LANES,), mean, _F32)

            def k_pass2(k, _):
                v = buf[r, pl.ds(k * LANES, LANES)]
                g = gam_v[pl.ds(k * LANES, LANES)]
                b = bet_v[pl.ds(k * LANES, LANES)]
                buf[r, pl.ds(k * LANES, LANES)] = (v - mvec) * (rstd * g) + b
                return 0

            lax.fori_loop(0, KD, k_pass2, 0)
            return 0

        lax.fori_loop(0, CH, row_body, 0)
        pltpu.sync_copy(buf, out_hbm.at[pl.ds(wbase + j * CH, CH)])
        return 0

    lax.fori_loop(0, NCHUNK, chunk_body, 0)


def kernel(input_ids, token_type_ids, token_table, pos_table, seg_table,
           gamma, beta):
    ids = input_ids.reshape(-1).astype(jnp.int32)
    tti = token_type_ids.reshape(-1).astype(jnp.int32)

    mesh = plsc.VectorSubcoreMesh(core_axis_name="c", subcore_axis_name="s")
    run = functools.partial(
        pl.kernel, mesh=mesh,
        compiler_params=pltpu.CompilerParams(needs_layout_passes=False),
        out_type=jax.ShapeDtypeStruct((N_TOK, D), _F32),
        scratch_types=[
            pltpu.VMEM((TPW,), jnp.int32),
            pltpu.VMEM((TPW,), jnp.int32),
            pltpu.VMEM((POS_STAGE, D), _F32),
            pltpu.VMEM((2, D), _F32),
            pltpu.VMEM((D,), _F32),
            pltpu.VMEM((D,), _F32),
            pltpu.VMEM((CH, D), _F32),
            pltpu.SemaphoreType.DMA,
        ],
    )(_body)
    out = run(ids, tti, token_table, pos_table, seg_table, gamma, beta)
    return out.reshape(BATCH, SEQ, D)
